# baseline (speedup 1.0000x reference)
import functools

import jax
import jax.numpy as jnp
from jax import lax
from jax.experimental import pallas as pl
from jax.experimental.pallas import tpu as pltpu
from jax.experimental.pallas import tpu_sc as plsc

_TOP_K = 2
_E = 64
_D = 768
_N = 32768
_T = 2048

_NW = 32
_C = _N // _NW
_L = 16


def _router_body(x_ref, wg_ref, logits_ref, meta_ref):
    x = x_ref[...]
    wg = wg_ref[...]
    logits = jax.lax.dot_general(
        wg, x, (((1,), (1,)), ((), ())), preferred_element_type=jnp.float32)
    logits_ref[...] = logits

    m = jnp.max(logits, axis=0, keepdims=True)
    e = jnp.exp(logits - m)
    p = e / jnp.sum(e, axis=0, keepdims=True)

    eiota = jax.lax.broadcasted_iota(jnp.int32, p.shape, 0).astype(jnp.float32)
    w1 = jnp.max(p, axis=0, keepdims=True)
    idx1 = jnp.min(jnp.where(p == w1, eiota, float(_E)), axis=0, keepdims=True)
    p2 = jnp.where(eiota == idx1, -1.0, p)
    w2 = jnp.max(p2, axis=0, keepdims=True)
    idx2 = jnp.min(jnp.where(p2 == w2, eiota, float(_E)), axis=0, keepdims=True)

    w1m = jnp.where(idx1 == float(_E - 1), 0.0, w1)
    w2m = jnp.where(idx2 == float(_E - 1), 0.0, w2)
    s = w1m + w2m
    w1n = w1m / s
    w2n = w2m / s

    meta_ref[...] = jnp.concatenate([idx1, idx2, w1n, w2n], axis=0)


def _tc_router(x, wg):
    n_blocks = _N // _T
    return pl.pallas_call(
        _router_body,
        grid=(n_blocks,),
        in_specs=[
            pl.BlockSpec((_T, _D), lambda i: (i, 0)),
            pl.BlockSpec((_E, _D), lambda i: (0, 0)),
        ],
        out_specs=[
            pl.BlockSpec((_E, _T), lambda i: (0, i)),
            pl.BlockSpec((4, _T), lambda i: (0, i)),
        ],
        out_shape=[
            jax.ShapeDtypeStruct((_E, _N), jnp.float32),
            jax.ShapeDtypeStruct((4, _N), jnp.float32),
        ],
    )(x, wg)


def _sc_scatter_body(meta_hbm, ew_hbm, meta_v, ew_v, sem):
    wid = lax.axis_index("s") * 2 + lax.axis_index("c")
    base = wid * _C
    meta_dma = [
        pltpu.async_copy(meta_hbm.at[k, pl.ds(base, _C)],
                         meta_v.at[pl.ds(k * _C, _C)], sem)
        for k in range(4)
    ]

    z16 = jnp.zeros((_L,), jnp.float32)
    _ZU = 64

    def _zero(i, _):
        b = i * (_L * _ZU)
        for k in range(_ZU):
            ew_v[pl.ds(b + k * _L, _L)] = z16
        return 0
    lax.fori_loop(0, (_E * _C) // (_L * _ZU), _zero, 0)

    for c in meta_dma:
        c.wait()

    lane = lax.iota(jnp.int32, _L)
    _SU = 8

    def _scatter(g, _):
        for k in range(_SU):
            o = (g * _SU + k) * _L
            t16 = o + lane
            e1 = meta_v[pl.ds(0 * _C + o, _L)].astype(jnp.int32)
            e2 = meta_v[pl.ds(1 * _C + o, _L)].astype(jnp.int32)
            w1 = meta_v[pl.ds(2 * _C + o, _L)]
            w2 = meta_v[pl.ds(3 * _C + o, _L)]
            plsc.store_scatter(ew_v, [e1 * _C + t16], w1)
            plsc.store_scatter(ew_v, [e2 * _C + t16], w2)
        return 0
    lax.fori_loop(0, _C // (_L * _SU), _scatter, 0)

    copies = [
        pltpu.async_copy(ew_v.at[pl.ds(r * _C, _C)],
                         ew_hbm.at[r, pl.ds(base, _C)], sem)
        for r in range(_E)
    ]
    for c in copies:
        c.wait()


_sc_scatter = functools.partial(
    pl.kernel,
    out_type=jax.ShapeDtypeStruct((_E, _N), jnp.float32),
    mesh=plsc.VectorSubcoreMesh(core_axis_name="c", subcore_axis_name="s"),
    compiler_params=pltpu.CompilerParams(needs_layout_passes=False),
    scratch_types=[
        pltpu.VMEM((4 * _C,), jnp.float32),
        pltpu.VMEM((_E * _C,), jnp.float32),
        pltpu.SemaphoreType.DMA,
    ],
)(_sc_scatter_body)


@jax.jit
def kernel(inputs, W_gate, W_pre):
    del W_pre
    x = inputs.astype(jnp.float32)
    logits_t, meta = _tc_router(x, W_gate)
    ew = _sc_scatter(meta)
    sel_idx = meta[0:2, :].T.astype(jnp.int32)
    sel_w = meta[2:4, :].T
    return (sel_idx, sel_w, ew, logits_t.T)


# speedup vs baseline: 1.0361x; 1.0361x over previous
import functools

import jax
import jax.numpy as jnp
from jax import lax
from jax.experimental import pallas as pl
from jax.experimental.pallas import tpu as pltpu
from jax.experimental.pallas import tpu_sc as plsc

_TOP_K = 2
_E = 64
_D = 768
_N = 32768
_T = 4096

_NW = 32
_C = _N // _NW
_L = 16


def _router_body(x_ref, wg_ref, logits_ref, meta_ref):
    x = x_ref[...]
    wg = wg_ref[...]
    logits = jax.lax.dot_general(
        wg, x, (((1,), (1,)), ((), ())), preferred_element_type=jnp.float32)
    logits_ref[...] = logits

    m = jnp.max(logits, axis=0, keepdims=True)
    e = jnp.exp(logits - m)
    p = e / jnp.sum(e, axis=0, keepdims=True)

    eiota = jax.lax.broadcasted_iota(jnp.int32, p.shape, 0).astype(jnp.float32)
    w1 = jnp.max(p, axis=0, keepdims=True)
    idx1 = jnp.min(jnp.where(p == w1, eiota, float(_E)), axis=0, keepdims=True)
    p2 = jnp.where(eiota == idx1, -1.0, p)
    w2 = jnp.max(p2, axis=0, keepdims=True)
    idx2 = jnp.min(jnp.where(p2 == w2, eiota, float(_E)), axis=0, keepdims=True)

    w1m = jnp.where(idx1 == float(_E - 1), 0.0, w1)
    w2m = jnp.where(idx2 == float(_E - 1), 0.0, w2)
    s = w1m + w2m
    w1n = w1m / s
    w2n = w2m / s

    meta_ref[...] = jnp.concatenate([idx1, idx2, w1n, w2n], axis=0)


def _tc_router(x, wg):
    n_blocks = _N // _T
    return pl.pallas_call(
        _router_body,
        grid=(n_blocks,),
        in_specs=[
            pl.BlockSpec((_T, _D), lambda i: (i, 0)),
            pl.BlockSpec((_E, _D), lambda i: (0, 0)),
        ],
        out_specs=[
            pl.BlockSpec((_E, _T), lambda i: (0, i)),
            pl.BlockSpec((4, _T), lambda i: (0, i)),
        ],
        out_shape=[
            jax.ShapeDtypeStruct((_E, _N), jnp.float32),
            jax.ShapeDtypeStruct((4, _N), jnp.float32),
        ],
    )(x, wg)


def _sc_scatter_body(meta_hbm, ew_hbm, meta_v, ew_v, sem):
    wid = lax.axis_index("s") * 2 + lax.axis_index("c")
    base = wid * _C
    meta_dma = [
        pltpu.async_copy(meta_hbm.at[k, pl.ds(base, _C)],
                         meta_v.at[pl.ds(k * _C, _C)], sem)
        for k in range(4)
    ]

    z16 = jnp.zeros((_L,), jnp.float32)
    _ZU = 64

    def _zero(i, _):
        r = i // (_C // (_L * _ZU))
        b = (i % (_C // (_L * _ZU))) * (_L * _ZU)
        for k in range(_ZU):
            ew_v[r, pl.ds(b + k * _L, _L)] = z16
        return 0
    lax.fori_loop(0, _E * _C // (_L * _ZU), _zero, 0)

    for c in meta_dma:
        c.wait()

    lane = lax.iota(jnp.int32, _L)
    _SU = 8

    def _scatter(g, _):
        for k in range(_SU):
            o = (g * _SU + k) * _L
            t16 = o + lane
            e1 = meta_v[pl.ds(0 * _C + o, _L)].astype(jnp.int32)
            e2 = meta_v[pl.ds(1 * _C + o, _L)].astype(jnp.int32)
            w1 = meta_v[pl.ds(2 * _C + o, _L)]
            w2 = meta_v[pl.ds(3 * _C + o, _L)]
            plsc.store_scatter(ew_v, [e1, t16], w1)
            plsc.store_scatter(ew_v, [e2, t16], w2)
        return 0
    lax.fori_loop(0, _C // (_L * _SU), _scatter, 0)

    pltpu.async_copy(ew_v, ew_hbm.at[:, pl.ds(base, _C)], sem).wait()


_sc_scatter = functools.partial(
    pl.kernel,
    out_type=jax.ShapeDtypeStruct((_E, _N), jnp.float32),
    mesh=plsc.VectorSubcoreMesh(core_axis_name="c", subcore_axis_name="s"),
    compiler_params=pltpu.CompilerParams(needs_layout_passes=False),
    scratch_types=[
        pltpu.VMEM((4 * _C,), jnp.float32),
        pltpu.VMEM((_E, _C), jnp.float32),
        pltpu.SemaphoreType.DMA,
    ],
)(_sc_scatter_body)


@jax.jit
def kernel(inputs, W_gate, W_pre):
    del W_pre
    x = inputs.astype(jnp.float32)
    logits_t, meta = _tc_router(x, W_gate)
    ew = _sc_scatter(meta)
    sel_idx = meta[0:2, :].T.astype(jnp.int32)
    sel_w = meta[2:4, :].T
    return (sel_idx, sel_w, ew, logits_t.T)


# speedup vs baseline: 1.0361x; 1.0001x over previous
import functools

import jax
import jax.numpy as jnp
from jax import lax
from jax.experimental import pallas as pl
from jax.experimental.pallas import tpu as pltpu
from jax.experimental.pallas import tpu_sc as plsc

_E = 64
_D = 768
_N = 32768
_T = 4096

_NW = 32
_C = _N // _NW
_L = 16


def _router_body(x_ref, wg_ref, logits_ref, meta_ref):
    x = x_ref[...]
    wg = wg_ref[...]
    logits = jax.lax.dot_general(
        wg, x, (((1,), (1,)), ((), ())), preferred_element_type=jnp.float32)
    logits_ref[...] = logits

    m = jnp.max(logits, axis=0, keepdims=True)
    e = jnp.exp(logits - m)
    p = e / jnp.sum(e, axis=0, keepdims=True)

    eiota = jax.lax.broadcasted_iota(jnp.int32, p.shape, 0).astype(jnp.float32)
    w1 = jnp.max(p, axis=0, keepdims=True)
    idx1 = jnp.min(jnp.where(p == w1, eiota, float(_E)), axis=0, keepdims=True)
    p2 = jnp.where(eiota == idx1, -1.0, p)
    w2 = jnp.max(p2, axis=0, keepdims=True)
    idx2 = jnp.min(jnp.where(p2 == w2, eiota, float(_E)), axis=0, keepdims=True)

    w1m = jnp.where(idx1 == float(_E - 1), 0.0, w1)
    w2m = jnp.where(idx2 == float(_E - 1), 0.0, w2)
    s = w1m + w2m
    w1n = w1m / s
    w2n = w2m / s

    meta_ref[...] = jnp.concatenate([idx1, idx2, w1n, w2n], axis=0)


def _tc_router(x, wg):
    n_blocks = _N // _T
    return pl.pallas_call(
        _router_body,
        grid=(n_blocks,),
        in_specs=[
            pl.BlockSpec((_T, _D), lambda i: (i, 0)),
            pl.BlockSpec((_E, _D), lambda i: (0, 0)),
        ],
        out_specs=[
            pl.BlockSpec((_E, _T), lambda i: (0, i)),
            pl.BlockSpec((4, _T), lambda i: (0, i)),
        ],
        out_shape=[
            jax.ShapeDtypeStruct((_E, _N), jnp.float32),
            jax.ShapeDtypeStruct((4, _N), jnp.float32),
        ],
    )(x, wg)


def _sc_scatter_body(meta_hbm, ew_hbm, meta_v, ew_v, sem):
    wid = lax.axis_index("s") * 2 + lax.axis_index("c")
    base = wid * _C
    meta_dma = [
        pltpu.async_copy(meta_hbm.at[k, pl.ds(base, _C)],
                         meta_v.at[pl.ds(k * _C, _C)], sem)
        for k in range(4)
    ]

    z16 = jnp.zeros((_L,), jnp.float32)
    _ZU = 64

    def _zero(i, _):
        r = i // (_C // (_L * _ZU))
        b = (i % (_C // (_L * _ZU))) * (_L * _ZU)
        for k in range(_ZU):
            ew_v[r, pl.ds(b + k * _L, _L)] = z16
        return 0
    lax.fori_loop(0, _E * _C // (_L * _ZU), _zero, 0)

    for c in meta_dma:
        c.wait()

    lane = lax.iota(jnp.int32, _L)
    _SU = 8

    def _scatter(g, _):
        for k in range(_SU):
            o = (g * _SU + k) * _L
            t16 = o + lane
            e1 = meta_v[pl.ds(0 * _C + o, _L)].astype(jnp.int32)
            e2 = meta_v[pl.ds(1 * _C + o, _L)].astype(jnp.int32)
            w1 = meta_v[pl.ds(2 * _C + o, _L)]
            w2 = meta_v[pl.ds(3 * _C + o, _L)]
            plsc.store_scatter(ew_v, [e1, t16], w1)
            plsc.store_scatter(ew_v, [e2, t16], w2)
        return 0
    lax.fori_loop(0, _C // (_L * _SU), _scatter, 0)

    pltpu.async_copy(ew_v, ew_hbm.at[:, pl.ds(base, _C)], sem).wait()


_sc_scatter = functools.partial(
    pl.kernel,
    out_type=jax.ShapeDtypeStruct((_E, _N), jnp.float32),
    mesh=plsc.VectorSubcoreMesh(core_axis_name="c", subcore_axis_name="s"),
    compiler_params=pltpu.CompilerParams(needs_layout_passes=False),
    scratch_types=[
        pltpu.VMEM((4 * _C,), jnp.float32),
        pltpu.VMEM((_E, _C), jnp.float32),
        pltpu.SemaphoreType.DMA,
    ],
)(_sc_scatter_body)


@jax.jit
def kernel(inputs, W_gate, W_pre):
    del W_pre
    x = inputs.astype(jnp.float32)
    logits_t, meta = _tc_router(x, W_gate)
    ew = _sc_scatter(meta)
    sel_idx = meta[0:2, :].T.astype(jnp.int32)
    sel_w = meta[2:4, :].T
    return (sel_idx, sel_w, ew, logits_t.T)
